# baseline (speedup 1.0000x reference)
import jax
import jax.numpy as jnp
from jax import lax
from jax.experimental import pallas as pl
from jax.experimental.pallas import tpu as pltpu
from jax.experimental.pallas import tpu_sc as plsc

N = 10000
E = 320000
D = 128

NC = 2
NS = 16
NW = NC * NS
FPT = D // NW
NP = 10240
IBKE = 8192
EP = 327680
NBLKE = EP // IBKE
NPAIR = NBLKE // 2
GU = 8
GPB = IBKE // 16 // GU
DSEG = EP // NW
DH = DSEG // 2
Bn = 1024


def _sc_body(xt_hbm, eidx_hbm, aggt_hbm, degp_hbm,
             xslab, aslab, dslab, ibufA, ibufB, siA, siB):
    c = lax.axis_index("c")
    s = lax.axis_index("s")
    wid = c * NS + s
    zeros16 = jnp.zeros((16,), jnp.float32)
    lo16 = jnp.full((16,), 0xFFFF, jnp.int32)
    npv = jnp.full((16,), NP, jnp.int32)

    def zacc(k, _):
        aslab[pl.ds(k * 16, 16)] = zeros16
        return 0
    lax.fori_loop(0, FPT * NP // 16, zacc, 0)

    def zdeg(i, _):
        dslab[pl.ds(i * 16, 16)] = zeros16
        return 0
    lax.fori_loop(0, NP // 16, zdeg, 0)

    pltpu.sync_copy(xt_hbm.at[wid], xslab)

    def run_block(ibuf):
        def grp(gi, _):
            for u in range(GU):
                base = (gi * GU + u) * 16
                w16 = ibuf[pl.ds(base, 16)]
                cidx = w16 & lo16
                ridx = lax.shift_right_logical(w16, 16)
                for f in range(FPT):
                    v = plsc.load_gather(xslab, [cidx])
                    plsc.addupdate_scatter(aslab, [ridx], v)
                    if f < FPT - 1:
                        cidx = cidx + npv
                        ridx = ridx + npv
            return 0
        lax.fori_loop(0, GPB, grp, 0)

    pltpu.async_copy(eidx_hbm.at[pl.ds(0, IBKE)], ibufA, siA)

    def pair(p, _):
        t = 2 * p
        pltpu.make_async_copy(eidx_hbm.at[pl.ds(0, IBKE)], ibufA, siA).wait()
        pltpu.async_copy(eidx_hbm.at[pl.ds((t + 1) * IBKE, IBKE)], ibufB, siB)
        run_block(ibufA)
        pltpu.make_async_copy(eidx_hbm.at[pl.ds(0, IBKE)], ibufB, siB).wait()
        @pl.when(p < NPAIR - 1)
        def _():
            pltpu.async_copy(eidx_hbm.at[pl.ds((t + 2) * IBKE, IBKE)],
                             ibufA, siA)
        run_block(ibufB)
        return 0
    lax.fori_loop(0, NPAIR, pair, 0)

    def deg_half(h):
        pltpu.sync_copy(eidx_hbm.at[pl.ds(wid * DSEG + h * DH, DH)],
                        ibufA.at[pl.ds(0, DH)])
        def dgrp(gi, _):
            for u in range(GU):
                base = (gi * GU + u) * 16
                w16 = ibufA[pl.ds(base, 16)]
                row16 = lax.shift_right_logical(w16, 16)
                plsc.addupdate_scatter(dslab, [row16], jnp.ones((16,), jnp.float32))
            return 0
        lax.fori_loop(0, DH // 16 // GU, dgrp, 0)
    deg_half(0)
    deg_half(1)

    pltpu.sync_copy(aslab, aggt_hbm.at[wid])
    pltpu.sync_copy(dslab, degp_hbm.at[wid])


def _sc_aggregate(xt3, eidx):
    mesh = plsc.VectorSubcoreMesh(
        core_axis_name="c", subcore_axis_name="s", num_cores=NC, num_subcores=NS
    )
    f32 = jnp.float32
    sem = pltpu.SemaphoreType.DMA
    return pl.kernel(
        _sc_body,
        out_type=[
            jax.ShapeDtypeStruct((NW, FPT * NP), f32),
            jax.ShapeDtypeStruct((NW, NP), f32),
        ],
        mesh=mesh,
        compiler_params=pltpu.CompilerParams(use_tc_tiling_on_sc=False,
                                             needs_layout_passes=False),
        scratch_types=[
            pltpu.VMEM((FPT * NP,), f32),
            pltpu.VMEM((FPT * NP,), f32),
            pltpu.VMEM((NP,), f32),
            pltpu.VMEM((IBKE,), jnp.int32),
            pltpu.VMEM((IBKE,), jnp.int32),
            sem, sem,
        ],
    )(xt3, eidx)


def _tc_body(a_ref, dp_ref, w_ref, b_ref, o_ref):
    a = a_ref[...].reshape(D, Bn)
    m = lax.dot_general(a, w_ref[...], (((0,), (1,)), ((), ())),
                        preferred_element_type=jnp.float32)
    deg = jnp.sum(dp_ref[...], axis=0).reshape(Bn, 1)
    dinv = jnp.where(deg > 0.0, 1.0 / deg, 0.0)
    o_ref[...] = m * dinv + b_ref[...]


def _tc_epilogue(aggt3, degp, W, b2):
    grid = (NP // Bn,)
    return pl.pallas_call(
        _tc_body,
        grid=grid,
        in_specs=[
            pl.BlockSpec((NW, FPT, Bn), lambda i: (0, 0, i)),
            pl.BlockSpec((NW, Bn), lambda i: (0, i)),
            pl.BlockSpec((D, D), lambda i: (0, 0)),
            pl.BlockSpec((1, D), lambda i: (0, 0)),
        ],
        out_specs=pl.BlockSpec((Bn, D), lambda i: (i, 0)),
        out_shape=jax.ShapeDtypeStruct((NP, D), jnp.float32),
    )(aggt3, degp, W, b2)


def kernel(x, edge_index, W, b):
    row = edge_index[0].astype(jnp.int32)
    col = edge_index[1].astype(jnp.int32)
    pad = EP - E
    sink = N + (jnp.arange(pad, dtype=jnp.int32) % (NP - N))
    rowp = jnp.concatenate([row, sink])
    colp = jnp.concatenate([col, jnp.zeros((pad,), jnp.int32)])
    eidx = (rowp << 16) | colp
    xp = jnp.concatenate([x, jnp.zeros((NP - N, D), x.dtype)], axis=0)
    xt3 = xp.T.reshape(NW, FPT * NP)

    aggt3, degp = _sc_aggregate(xt3, eidx)
    aggt3 = aggt3.reshape(NW, FPT, NP)

    out = _tc_epilogue(aggt3, degp, W, b.reshape(1, D))
    return out[:N]


# speedup vs baseline: 2.9325x; 2.9325x over previous
import jax
import jax.numpy as jnp
from jax import lax
from jax.experimental import pallas as pl
from jax.experimental.pallas import tpu as pltpu
from jax.experimental.pallas import tpu_sc as plsc

N = 10000
E = 320000
D = 128

NC = 2
NS = 16
CHUNK = 128
Dh = D // NC
NCH = 160
IBLK = 16
NBLK = NCH // IBLK
NPAIR = NBLK // 2
EP = NS * NCH * CHUNK
NP = 10240
STRIPE = NP // NS


def _sc_body(x0_hbm, x1_hbm, row_hbm, col_hbm,
             agg0_hbm, agg1_hbm, deg0_hbm, deg1_hbm,
             xs_s, agg_s, deg_s,
             rbA, rbB, cbA, cbB, g0, g1, g2, g3, zbuf, ones_v,
             sg0, sg1, sg2, sg3, ss0, ss1, ss2, ss3, siA, siB, sdA, sdB):
    c = lax.axis_index("c")
    s = lax.axis_index("s")
    gb = [g0, g1, g2, g3]
    sg = [sg0, sg1, sg2, sg3]
    ss = [ss0, ss1, ss2, ss3]
    zeros16 = jnp.zeros((16,), jnp.float32)
    ones16 = jnp.ones((16,), jnp.float32)

    def zrow(r, _):
        def zcol(k, _):
            g0[r, pl.ds(k * 16, 16)] = zeros16
            return 0
        return lax.fori_loop(0, Dh // 16, zcol, 0)
    lax.fori_loop(0, CHUNK, zrow, 0)

    def z1(i, _):
        zbuf[pl.ds(i * 16, 16)] = zeros16
        ones_v[pl.ds(i * 16, 16)] = ones16
        return 0
    lax.fori_loop(0, CHUNK // 16, z1, 0)

    def zagg(k, _):
        pltpu.sync_copy(g0, agg_s.at[pl.ds(s * STRIPE + k * CHUNK, CHUNK)])
        pltpu.sync_copy(zbuf, deg_s.at[pl.ds(s * STRIPE + k * CHUNK, CHUNK)])
        return 0
    lax.fori_loop(0, STRIPE // CHUNK, zagg, 0)

    @pl.when(c == 0)
    def _():
        pltpu.sync_copy(x0_hbm.at[pl.ds(s * STRIPE, STRIPE)],
                        xs_s.at[pl.ds(s * STRIPE, STRIPE)])

    @pl.when(c == 1)
    def _():
        pltpu.sync_copy(x1_hbm.at[pl.ds(s * STRIPE, STRIPE)],
                        xs_s.at[pl.ds(s * STRIPE, STRIPE)])

    pltpu.sync_copy(row_hbm.at[s, pl.ds(0, IBLK)], rbA)
    pltpu.sync_copy(col_hbm.at[s, pl.ds(0, IBLK)], cbA)

    plsc.subcore_barrier()

    def chunk(l, rb, cb, cbn, first_ever):
        k = l % 4
        kp = (k + 3) % 4
        pltpu.make_async_copy(xs_s.at[cb.at[l]], gb[k], sg[k]).wait()
        pltpu.async_copy(gb[k], agg_s.at[rb.at[l]], ss[k], add=True)

        dsem = sdA if k < 2 else sdB
        @pl.when(c == (k % 2))
        def _():
            if not (first_ever and l < 4):
                pltpu.make_async_copy(ones_v, deg_s.at[rb.at[l]], dsem).wait()
            pltpu.async_copy(ones_v, deg_s.at[rb.at[l]], dsem, add=True)

        if not (first_ever and l == 0):
            pltpu.make_async_copy(gb[kp], agg_s.at[rb.at[l]], ss[kp]).wait()
        if l + 3 < IBLK:
            pltpu.async_copy(xs_s.at[cb.at[l + 3]], gb[kp], sg[kp])
        else:
            pltpu.async_copy(xs_s.at[cbn.at[l + 3 - IBLK]], gb[kp], sg[kp])

    def block(t, rb, cb, rbn, cbn, sin, first_ever):
        for l in range(4):
            chunk(l, rb, cb, cbn, first_ever)
        off = lax.rem(t + 1, NBLK) * IBLK
        pltpu.async_copy(row_hbm.at[s, pl.ds(off, IBLK)], rbn, sin)
        pltpu.async_copy(col_hbm.at[s, pl.ds(off, IBLK)], cbn, sin)
        for l in range(4, 12):
            chunk(l, rb, cb, cbn, first_ever)
        pltpu.make_async_copy(row_hbm.at[s, pl.ds(0, IBLK)], rbn, sin).wait()
        pltpu.make_async_copy(col_hbm.at[s, pl.ds(0, IBLK)], cbn, sin).wait()
        for l in range(12, 16):
            chunk(l, rb, cb, cbn, first_ever)

    for k in range(3):
        pltpu.async_copy(xs_s.at[cbA.at[k]], gb[k], sg[k])

    block(jnp.int32(0), rbA, cbA, rbB, cbB, siB, True)
    block(jnp.int32(1), rbB, cbB, rbA, cbA, siA, False)

    def pair(p, _):
        t = 2 * p
        block(t, rbA, cbA, rbB, cbB, siB, False)
        block(t + 1, rbB, cbB, rbA, cbA, siA, False)
        return 0
    lax.fori_loop(1, NPAIR, pair, 0)

    pltpu.make_async_copy(gb[3], agg_s.at[rbA.at[0]], ss[3]).wait()
    for k in range(3):
        pltpu.make_async_copy(xs_s.at[cbA.at[0]], gb[k], sg[k]).wait()
    pltpu.make_async_copy(ones_v, deg_s.at[rbA.at[0]], sdA).wait()
    pltpu.make_async_copy(ones_v, deg_s.at[rbA.at[0]], sdB).wait()

    plsc.subcore_barrier()

    @pl.when(c == 0)
    def _():
        pltpu.sync_copy(agg_s.at[pl.ds(s * STRIPE, STRIPE)],
                        agg0_hbm.at[pl.ds(s * STRIPE, STRIPE)])

    @pl.when(c == 1)
    def _():
        pltpu.sync_copy(agg_s.at[pl.ds(s * STRIPE, STRIPE)],
                        agg1_hbm.at[pl.ds(s * STRIPE, STRIPE)])

    @pl.when((c == 0) & (s == 0))
    def _():
        pltpu.sync_copy(deg_s, deg0_hbm)

    @pl.when((c == 1) & (s == 0))
    def _():
        pltpu.sync_copy(deg_s, deg1_hbm)


def _sc_aggregate(x0, x1, row_r, col_r):
    mesh = plsc.VectorSubcoreMesh(
        core_axis_name="c", subcore_axis_name="s", num_cores=NC, num_subcores=NS
    )
    f32 = jnp.float32
    sem = pltpu.SemaphoreType.DMA
    return pl.kernel(
        _sc_body,
        out_type=[
            jax.ShapeDtypeStruct((NP, Dh), f32),
            jax.ShapeDtypeStruct((NP, Dh), f32),
            jax.ShapeDtypeStruct((NP,), f32),
            jax.ShapeDtypeStruct((NP,), f32),
        ],
        mesh=mesh,
        compiler_params=pltpu.CompilerParams(use_tc_tiling_on_sc=False),
        scratch_types=[
            pltpu.VMEM_SHARED((NP, Dh), f32),
            pltpu.VMEM_SHARED((NP, Dh), f32),
            pltpu.VMEM_SHARED((NP,), f32),
            pltpu.VMEM((IBLK, CHUNK), jnp.int32),
            pltpu.VMEM((IBLK, CHUNK), jnp.int32),
            pltpu.VMEM((IBLK, CHUNK), jnp.int32),
            pltpu.VMEM((IBLK, CHUNK), jnp.int32),
            pltpu.VMEM((CHUNK, Dh), f32),
            pltpu.VMEM((CHUNK, Dh), f32),
            pltpu.VMEM((CHUNK, Dh), f32),
            pltpu.VMEM((CHUNK, Dh), f32),
            pltpu.VMEM((CHUNK,), f32),
            pltpu.VMEM((CHUNK,), f32),
            sem, sem, sem, sem,
            sem, sem, sem, sem,
            sem, sem,
            sem, sem,
        ],
    )(x0, x1, row_r, col_r)


def _tc_body(a0_ref, a1_ref, d0_ref, d1_ref, w0_ref, w1_ref, b_ref, o_ref):
    deg = d0_ref[...] + d1_ref[...]
    dinv = jnp.where(deg > 0.0, 1.0 / deg, 0.0)
    a0 = a0_ref[...] * dinv
    a1 = a1_ref[...] * dinv
    o_ref[...] = (
        jnp.dot(a0, w0_ref[...], preferred_element_type=jnp.float32)
        + jnp.dot(a1, w1_ref[...], preferred_element_type=jnp.float32)
        + b_ref[...]
    )


def _tc_epilogue(agg0, agg1, deg0, deg1, W0, W1, b2):
    Bn = 1024
    grid = (NP // Bn,)
    return pl.pallas_call(
        _tc_body,
        grid=grid,
        in_specs=[
            pl.BlockSpec((Bn, Dh), lambda i: (i, 0)),
            pl.BlockSpec((Bn, Dh), lambda i: (i, 0)),
            pl.BlockSpec((Bn, 1), lambda i: (i, 0)),
            pl.BlockSpec((Bn, 1), lambda i: (i, 0)),
            pl.BlockSpec((Dh, D), lambda i: (0, 0)),
            pl.BlockSpec((Dh, D), lambda i: (0, 0)),
            pl.BlockSpec((1, D), lambda i: (0, 0)),
        ],
        out_specs=pl.BlockSpec((Bn, D), lambda i: (i, 0)),
        out_shape=jax.ShapeDtypeStruct((NP, D), jnp.float32),
    )(agg0, agg1, deg0, deg1, W0, W1, b2)


def kernel(x, edge_index, W, b):
    row = edge_index[0].astype(jnp.int32)
    col = edge_index[1].astype(jnp.int32)
    pad = EP - E
    sink = N + (jnp.arange(pad, dtype=jnp.int32) % (NP - N))
    rowp = jnp.concatenate([row, sink])
    colp = jnp.concatenate([col, jnp.zeros((pad,), jnp.int32)])
    row_r = rowp.reshape(NS, NCH, CHUNK)
    col_r = colp.reshape(NS, NCH, CHUNK)
    xp = jnp.concatenate([x, jnp.zeros((NP - N, D), x.dtype)], axis=0)
    x0 = xp[:, :Dh]
    x1 = xp[:, Dh:]

    agg0, agg1, deg0, deg1 = _sc_aggregate(x0, x1, row_r, col_r)

    W0 = W[:, :Dh].T
    W1 = W[:, Dh:].T
    out = _tc_epilogue(agg0, agg1, deg0.reshape(NP, 1), deg1.reshape(NP, 1),
                       W0, W1, b.reshape(1, D))
    return out[:N]


# speedup vs baseline: 3.1861x; 1.0865x over previous
import jax
import jax.numpy as jnp
from jax import lax
from jax.experimental import pallas as pl
from jax.experimental.pallas import tpu as pltpu
from jax.experimental.pallas import tpu_sc as plsc

N = 10000
E = 320000
D = 128

NC = 2
NS = 16
CHUNK = 128
Dh = D // NC
NCH = 160
IBLK = 16
NBLK = NCH // IBLK
NPAIR = NBLK // 2
EP = NS * NCH * CHUNK
NP = 10240
STRIPE = NP // NS


def _sc_body(x_hbm, row_hbm, col_hbm,
             agg0_hbm, agg1_hbm, deg0_hbm, deg1_hbm,
             xs_s, agg_s, deg_s,
             rbA, rbB, cbA, cbB, g0, g1, g2, g3, zbuf, ones_v,
             sg0, sg1, sg2, sg3, ss0, ss1, ss2, ss3, siA, siB, sdA, sdB):
    c = lax.axis_index("c")
    s = lax.axis_index("s")
    gb = [g0, g1, g2, g3]
    sg = [sg0, sg1, sg2, sg3]
    ss = [ss0, ss1, ss2, ss3]
    zeros16 = jnp.zeros((16,), jnp.float32)
    ones16 = jnp.ones((16,), jnp.float32)

    def zrow(r, _):
        def zcol(k, _):
            g0[r, pl.ds(k * 16, 16)] = zeros16
            return 0
        return lax.fori_loop(0, Dh // 16, zcol, 0)
    lax.fori_loop(0, CHUNK, zrow, 0)

    def z1(i, _):
        zbuf[pl.ds(i * 16, 16)] = zeros16
        ones_v[pl.ds(i * 16, 16)] = ones16
        return 0
    lax.fori_loop(0, CHUNK // 16, z1, 0)

    def zagg(k, _):
        pltpu.sync_copy(g0, agg_s.at[pl.ds(s * STRIPE + k * CHUNK, CHUNK)])
        pltpu.sync_copy(zbuf, deg_s.at[pl.ds(s * STRIPE + k * CHUNK, CHUNK)])
        return 0
    lax.fori_loop(0, STRIPE // CHUNK, zagg, 0)

    @pl.when((s < 10) & (c == 0))
    def _():
        pltpu.sync_copy(x_hbm.at[pl.ds(s * 1000, 1000), pl.ds(0, Dh)],
                        xs_s.at[pl.ds(s * 1000, 1000)])

    @pl.when((s < 10) & (c == 1))
    def _():
        pltpu.sync_copy(x_hbm.at[pl.ds(s * 1000, 1000), pl.ds(Dh, Dh)],
                        xs_s.at[pl.ds(s * 1000, 1000)])

    pltpu.sync_copy(row_hbm.at[s, pl.ds(0, IBLK)], rbA)
    pltpu.sync_copy(col_hbm.at[s, pl.ds(0, IBLK)], cbA)

    plsc.subcore_barrier()

    def chunk(l, rb, cb, cbn, first_ever):
        k = l % 4
        kp = (k + 3) % 4
        pltpu.make_async_copy(xs_s.at[cb.at[l]], gb[k], sg[k]).wait()
        pltpu.async_copy(gb[k], agg_s.at[rb.at[l]], ss[k], add=True)

        dsem = sdA if k < 2 else sdB
        @pl.when(c == (k % 2))
        def _():
            if not (first_ever and l < 4):
                pltpu.make_async_copy(ones_v, deg_s.at[rb.at[l]], dsem).wait()
            pltpu.async_copy(ones_v, deg_s.at[rb.at[l]], dsem, add=True)

        if not (first_ever and l == 0):
            pltpu.make_async_copy(gb[kp], agg_s.at[rb.at[l]], ss[kp]).wait()
        if l + 3 < IBLK:
            pltpu.async_copy(xs_s.at[cb.at[l + 3]], gb[kp], sg[kp])
        else:
            pltpu.async_copy(xs_s.at[cbn.at[l + 3 - IBLK]], gb[kp], sg[kp])

    def block(t, rb, cb, rbn, cbn, sin, first_ever):
        for l in range(4):
            chunk(l, rb, cb, cbn, first_ever)
        off = lax.rem(t + 1, NBLK) * IBLK
        pltpu.async_copy(row_hbm.at[s, pl.ds(off, IBLK)], rbn, sin)
        pltpu.async_copy(col_hbm.at[s, pl.ds(off, IBLK)], cbn, sin)
        for l in range(4, 12):
            chunk(l, rb, cb, cbn, first_ever)
        pltpu.make_async_copy(row_hbm.at[s, pl.ds(0, IBLK)], rbn, sin).wait()
        pltpu.make_async_copy(col_hbm.at[s, pl.ds(0, IBLK)], cbn, sin).wait()
        for l in range(12, 16):
            chunk(l, rb, cb, cbn, first_ever)

    for k in range(3):
        pltpu.async_copy(xs_s.at[cbA.at[k]], gb[k], sg[k])

    block(jnp.int32(0), rbA, cbA, rbB, cbB, siB, True)
    block(jnp.int32(1), rbB, cbB, rbA, cbA, siA, False)

    def pair(p, _):
        t = 2 * p
        block(t, rbA, cbA, rbB, cbB, siB, False)
        block(t + 1, rbB, cbB, rbA, cbA, siA, False)
        return 0
    lax.fori_loop(1, NPAIR, pair, 0)

    pltpu.make_async_copy(gb[3], agg_s.at[rbA.at[0]], ss[3]).wait()
    for k in range(3):
        pltpu.make_async_copy(xs_s.at[cbA.at[0]], gb[k], sg[k]).wait()
    pltpu.make_async_copy(ones_v, deg_s.at[rbA.at[0]], sdA).wait()
    pltpu.make_async_copy(ones_v, deg_s.at[rbA.at[0]], sdB).wait()

    plsc.subcore_barrier()

    @pl.when(c == 0)
    def _():
        pltpu.sync_copy(agg_s.at[pl.ds(s * STRIPE, STRIPE)],
                        agg0_hbm.at[pl.ds(s * STRIPE, STRIPE)])

    @pl.when(c == 1)
    def _():
        pltpu.sync_copy(agg_s.at[pl.ds(s * STRIPE, STRIPE)],
                        agg1_hbm.at[pl.ds(s * STRIPE, STRIPE)])

    @pl.when((c == 0) & (s == 0))
    def _():
        pltpu.sync_copy(deg_s, deg0_hbm)

    @pl.when((c == 1) & (s == 0))
    def _():
        pltpu.sync_copy(deg_s, deg1_hbm)


def _sc_aggregate(x, row_r, col_r):
    mesh = plsc.VectorSubcoreMesh(
        core_axis_name="c", subcore_axis_name="s", num_cores=NC, num_subcores=NS
    )
    f32 = jnp.float32
    sem = pltpu.SemaphoreType.DMA
    return pl.kernel(
        _sc_body,
        out_type=[
            jax.ShapeDtypeStruct((NP, Dh), f32),
            jax.ShapeDtypeStruct((NP, Dh), f32),
            jax.ShapeDtypeStruct((NP,), f32),
            jax.ShapeDtypeStruct((NP,), f32),
        ],
        mesh=mesh,
        compiler_params=pltpu.CompilerParams(use_tc_tiling_on_sc=False),
        scratch_types=[
            pltpu.VMEM_SHARED((NP, Dh), f32),
            pltpu.VMEM_SHARED((NP, Dh), f32),
            pltpu.VMEM_SHARED((NP,), f32),
            pltpu.VMEM((IBLK, CHUNK), jnp.int32),
            pltpu.VMEM((IBLK, CHUNK), jnp.int32),
            pltpu.VMEM((IBLK, CHUNK), jnp.int32),
            pltpu.VMEM((IBLK, CHUNK), jnp.int32),
            pltpu.VMEM((CHUNK, Dh), f32),
            pltpu.VMEM((CHUNK, Dh), f32),
            pltpu.VMEM((CHUNK, Dh), f32),
            pltpu.VMEM((CHUNK, Dh), f32),
            pltpu.VMEM((CHUNK,), f32),
            pltpu.VMEM((CHUNK,), f32),
            sem, sem, sem, sem,
            sem, sem, sem, sem,
            sem, sem,
            sem, sem,
        ],
    )(x, row_r, col_r)


def _tc_body(a0_ref, a1_ref, d0_ref, d1_ref, w0_ref, w1_ref, b_ref, o_ref):
    deg = d0_ref[...] + d1_ref[...]
    dinv = jnp.where(deg > 0.0, 1.0 / deg, 0.0)
    a0 = a0_ref[...] * dinv
    a1 = a1_ref[...] * dinv
    o_ref[...] = (
        jnp.dot(a0, w0_ref[...], preferred_element_type=jnp.float32)
        + jnp.dot(a1, w1_ref[...], preferred_element_type=jnp.float32)
        + b_ref[...]
    )


def _tc_epilogue(agg0, agg1, deg0, deg1, W0, W1, b2):
    Bn = 1000
    grid = (N // Bn,)
    return pl.pallas_call(
        _tc_body,
        grid=grid,
        in_specs=[
            pl.BlockSpec((Bn, Dh), lambda i: (i, 0)),
            pl.BlockSpec((Bn, Dh), lambda i: (i, 0)),
            pl.BlockSpec((Bn, 1), lambda i: (i, 0)),
            pl.BlockSpec((Bn, 1), lambda i: (i, 0)),
            pl.BlockSpec((Dh, D), lambda i: (0, 0)),
            pl.BlockSpec((Dh, D), lambda i: (0, 0)),
            pl.BlockSpec((1, D), lambda i: (0, 0)),
        ],
        out_specs=pl.BlockSpec((Bn, D), lambda i: (i, 0)),
        out_shape=jax.ShapeDtypeStruct((N, D), jnp.float32),
    )(agg0, agg1, deg0, deg1, W0, W1, b2)


def kernel(x, edge_index, W, b):
    row = edge_index[0].astype(jnp.int32)
    col = edge_index[1].astype(jnp.int32)
    pad = EP - E
    sink = N + (jnp.arange(pad, dtype=jnp.int32) % (NP - N))
    rowp = jnp.concatenate([row, sink])
    colp = jnp.concatenate([col, jnp.zeros((pad,), jnp.int32)])
    row_r = rowp.reshape(NS, NCH, CHUNK)
    col_r = colp.reshape(NS, NCH, CHUNK)
    agg0, agg1, deg0, deg1 = _sc_aggregate(x, row_r, col_r)

    W0 = W[:, :Dh].T
    W1 = W[:, Dh:].T
    return _tc_epilogue(agg0, agg1, deg0.reshape(NP, 1), deg1.reshape(NP, 1),
                        W0, W1, b.reshape(1, D))


# speedup vs baseline: 3.2499x; 1.0200x over previous
import jax
import jax.numpy as jnp
from jax import lax
from jax.experimental import pallas as pl
from jax.experimental.pallas import tpu as pltpu
from jax.experimental.pallas import tpu_sc as plsc

N = 10000
E = 320000
D = 128

NC = 2
NS = 16
CHUNK = 128
Dh = D // NC
NCH = 160
IBLK = 16
NBLK = NCH // IBLK
NPAIR = NBLK // 2
EP = NS * NCH * CHUNK
NP = 10240
STRIPE = NP // NS


def _sc_body(x_hbm, row_hbm, col_hbm,
             agg0_hbm, agg1_hbm, deg0_hbm, deg1_hbm,
             xs_s, agg_s, deg_s,
             rbA, rbB, cbA, cbB, g0, g1, g2, g3, zbuf, ones_v,
             sg0, sg1, sg2, sg3, ss0, ss1, ss2, ss3, siA, siB, sdA, sdB):
    c = lax.axis_index("c")
    s = lax.axis_index("s")
    gb = [g0, g1, g2, g3]
    sg = [sg0, sg1, sg2, sg3]
    ss = [ss0, ss1, ss2, ss3]
    zeros16 = jnp.zeros((16,), jnp.float32)
    ones16 = jnp.ones((16,), jnp.float32)

    @pl.when((s < 10) & (c == 0))
    def _():
        pltpu.async_copy(x_hbm.at[pl.ds(s * 1000, 1000), pl.ds(0, Dh)],
                         xs_s.at[pl.ds(s * 1000, 1000)], sdA)

    @pl.when((s < 10) & (c == 1))
    def _():
        pltpu.async_copy(x_hbm.at[pl.ds(s * 1000, 1000), pl.ds(Dh, Dh)],
                         xs_s.at[pl.ds(s * 1000, 1000)], sdA)

    pltpu.async_copy(row_hbm.at[s, pl.ds(0, IBLK)], rbA, sdB)
    pltpu.async_copy(col_hbm.at[s, pl.ds(0, IBLK)], cbA, sdB)

    def zrow(r, _):
        def zcol(k, _):
            g0[r, pl.ds(k * 16, 16)] = zeros16
            return 0
        return lax.fori_loop(0, Dh // 16, zcol, 0)
    lax.fori_loop(0, CHUNK, zrow, 0)

    def z1(i, _):
        zbuf[pl.ds(i * 16, 16)] = zeros16
        ones_v[pl.ds(i * 16, 16)] = ones16
        return 0
    lax.fori_loop(0, CHUNK // 16, z1, 0)

    def zagg(k, _):
        pltpu.sync_copy(g0, agg_s.at[pl.ds(s * STRIPE + k * CHUNK, CHUNK)])
        pltpu.sync_copy(zbuf, deg_s.at[pl.ds(s * STRIPE + k * CHUNK, CHUNK)])
        return 0
    lax.fori_loop(0, STRIPE // CHUNK, zagg, 0)

    @pl.when((s < 10) & (c == 0))
    def _():
        pltpu.make_async_copy(x_hbm.at[pl.ds(s * 1000, 1000), pl.ds(0, Dh)],
                              xs_s.at[pl.ds(s * 1000, 1000)], sdA).wait()

    @pl.when((s < 10) & (c == 1))
    def _():
        pltpu.make_async_copy(x_hbm.at[pl.ds(s * 1000, 1000), pl.ds(Dh, Dh)],
                              xs_s.at[pl.ds(s * 1000, 1000)], sdA).wait()

    pltpu.make_async_copy(row_hbm.at[s, pl.ds(0, IBLK)], rbA, sdB).wait()
    pltpu.make_async_copy(col_hbm.at[s, pl.ds(0, IBLK)], cbA, sdB).wait()

    plsc.subcore_barrier()

    def chunk(l, rb, cb, cbn, first_ever):
        k = l % 4
        kp = (k + 3) % 4
        pltpu.make_async_copy(xs_s.at[cb.at[l]], gb[k], sg[k]).wait()
        pltpu.async_copy(gb[k], agg_s.at[rb.at[l]], ss[k], add=True)

        dsem = sdA if k < 2 else sdB
        @pl.when(c == (k % 2))
        def _():
            if not (first_ever and l < 4):
                pltpu.make_async_copy(ones_v, deg_s.at[rb.at[l]], dsem).wait()
            pltpu.async_copy(ones_v, deg_s.at[rb.at[l]], dsem, add=True)

        if not (first_ever and l == 0):
            pltpu.make_async_copy(gb[kp], agg_s.at[rb.at[l]], ss[kp]).wait()
        if l + 3 < IBLK:
            pltpu.async_copy(xs_s.at[cb.at[l + 3]], gb[kp], sg[kp])
        else:
            pltpu.async_copy(xs_s.at[cbn.at[l + 3 - IBLK]], gb[kp], sg[kp])

    def block(t, rb, cb, rbn, cbn, sin, first_ever):
        for l in range(4):
            chunk(l, rb, cb, cbn, first_ever)
        off = lax.rem(t + 1, NBLK) * IBLK
        pltpu.async_copy(row_hbm.at[s, pl.ds(off, IBLK)], rbn, sin)
        pltpu.async_copy(col_hbm.at[s, pl.ds(off, IBLK)], cbn, sin)
        for l in range(4, 12):
            chunk(l, rb, cb, cbn, first_ever)
        pltpu.make_async_copy(row_hbm.at[s, pl.ds(0, IBLK)], rbn, sin).wait()
        pltpu.make_async_copy(col_hbm.at[s, pl.ds(0, IBLK)], cbn, sin).wait()
        for l in range(12, 16):
            chunk(l, rb, cb, cbn, first_ever)

    for k in range(3):
        pltpu.async_copy(xs_s.at[cbA.at[k]], gb[k], sg[k])

    block(jnp.int32(0), rbA, cbA, rbB, cbB, siB, True)
    block(jnp.int32(1), rbB, cbB, rbA, cbA, siA, False)

    def pair(p, _):
        t = 2 * p
        block(t, rbA, cbA, rbB, cbB, siB, False)
        block(t + 1, rbB, cbB, rbA, cbA, siA, False)
        return 0
    lax.fori_loop(1, NPAIR, pair, 0)

    pltpu.make_async_copy(gb[3], agg_s.at[rbA.at[0]], ss[3]).wait()
    for k in range(3):
        pltpu.make_async_copy(xs_s.at[cbA.at[0]], gb[k], sg[k]).wait()
    pltpu.make_async_copy(ones_v, deg_s.at[rbA.at[0]], sdA).wait()
    pltpu.make_async_copy(ones_v, deg_s.at[rbA.at[0]], sdB).wait()

    plsc.subcore_barrier()

    @pl.when(c == 0)
    def _():
        pltpu.sync_copy(agg_s.at[pl.ds(s * STRIPE, STRIPE)],
                        agg0_hbm.at[pl.ds(s * STRIPE, STRIPE)])

    @pl.when(c == 1)
    def _():
        pltpu.sync_copy(agg_s.at[pl.ds(s * STRIPE, STRIPE)],
                        agg1_hbm.at[pl.ds(s * STRIPE, STRIPE)])

    @pl.when((c == 0) & (s == 0))
    def _():
        pltpu.sync_copy(deg_s, deg0_hbm)

    @pl.when((c == 1) & (s == 0))
    def _():
        pltpu.sync_copy(deg_s, deg1_hbm)


def _sc_aggregate(x, row_r, col_r):
    mesh = plsc.VectorSubcoreMesh(
        core_axis_name="c", subcore_axis_name="s", num_cores=NC, num_subcores=NS
    )
    f32 = jnp.float32
    sem = pltpu.SemaphoreType.DMA
    return pl.kernel(
        _sc_body,
        out_type=[
            jax.ShapeDtypeStruct((NP, Dh), f32),
            jax.ShapeDtypeStruct((NP, Dh), f32),
            jax.ShapeDtypeStruct((NP,), f32),
            jax.ShapeDtypeStruct((NP,), f32),
        ],
        mesh=mesh,
        compiler_params=pltpu.CompilerParams(use_tc_tiling_on_sc=False),
        scratch_types=[
            pltpu.VMEM_SHARED((NP, Dh), f32),
            pltpu.VMEM_SHARED((NP, Dh), f32),
            pltpu.VMEM_SHARED((NP,), f32),
            pltpu.VMEM((IBLK, CHUNK), jnp.int32),
            pltpu.VMEM((IBLK, CHUNK), jnp.int32),
            pltpu.VMEM((IBLK, CHUNK), jnp.int32),
            pltpu.VMEM((IBLK, CHUNK), jnp.int32),
            pltpu.VMEM((CHUNK, Dh), f32),
            pltpu.VMEM((CHUNK, Dh), f32),
            pltpu.VMEM((CHUNK, Dh), f32),
            pltpu.VMEM((CHUNK, Dh), f32),
            pltpu.VMEM((CHUNK,), f32),
            pltpu.VMEM((CHUNK,), f32),
            sem, sem, sem, sem,
            sem, sem, sem, sem,
            sem, sem,
            sem, sem,
        ],
    )(x, row_r, col_r)


def _tc_body(a0_ref, a1_ref, d0_ref, d1_ref, w_ref, b_ref, o_ref):
    deg = d0_ref[...] + d1_ref[...]
    dinv = jnp.where(deg > 0.0, 1.0 / deg, 0.0)
    a0 = a0_ref[...] * dinv
    a1 = a1_ref[...] * dinv
    w = w_ref[...]
    dn = (((1,), (1,)), ((), ()))
    o_ref[...] = (
        lax.dot_general(a0, w[:, :Dh], dn, preferred_element_type=jnp.float32)
        + lax.dot_general(a1, w[:, Dh:], dn, preferred_element_type=jnp.float32)
        + b_ref[...]
    )


def _tc_epilogue(agg0, agg1, deg0, deg1, W, b2):
    Bn = 1000
    grid = (N // Bn,)
    return pl.pallas_call(
        _tc_body,
        grid=grid,
        in_specs=[
            pl.BlockSpec((Bn, Dh), lambda i: (i, 0)),
            pl.BlockSpec((Bn, Dh), lambda i: (i, 0)),
            pl.BlockSpec((Bn, 1), lambda i: (i, 0)),
            pl.BlockSpec((Bn, 1), lambda i: (i, 0)),
            pl.BlockSpec((D, D), lambda i: (0, 0)),
            pl.BlockSpec((1, D), lambda i: (0, 0)),
        ],
        out_specs=pl.BlockSpec((Bn, D), lambda i: (i, 0)),
        out_shape=jax.ShapeDtypeStruct((N, D), jnp.float32),
    )(agg0, agg1, deg0, deg1, W, b2)


def kernel(x, edge_index, W, b):
    row = edge_index[0].astype(jnp.int32)
    col = edge_index[1].astype(jnp.int32)
    pad = EP - E
    sink = N + (jnp.arange(pad, dtype=jnp.int32) % (NP - N))
    rowp = jnp.concatenate([row, sink])
    colp = jnp.concatenate([col, jnp.zeros((pad,), jnp.int32)])
    row_r = rowp.reshape(NS, NCH, CHUNK)
    col_r = colp.reshape(NS, NCH, CHUNK)
    agg0, agg1, deg0, deg1 = _sc_aggregate(x, row_r, col_r)

    return _tc_epilogue(agg0, agg1, deg0.reshape(NP, 1), deg1.reshape(NP, 1),
                        W, b.reshape(1, D))


# speedup vs baseline: 3.3521x; 1.0315x over previous
import jax
import jax.numpy as jnp
from jax import lax
from jax.experimental import pallas as pl
from jax.experimental.pallas import tpu as pltpu
from jax.experimental.pallas import tpu_sc as plsc

N = 10000
E = 320000
D = 128

NC = 2
NS = 16
CHUNK = 128
Dh = D // NC
NCHT = 156
IBLK = 26
NBLK = NCHT // IBLK
NPAIR = NBLK // 2
ECH = E // CHUNK
NP = 10240
STRIPE = NP // NS


def _sc_body(x_hbm, row_hbm, col_hbm,
             agg0_hbm, agg1_hbm, deg0_hbm, deg1_hbm,
             xs_s, agg_s, deg_s,
             rbA, rbB, cbA, cbB, g0, g1, g2, g3, zbuf, ones_v,
             sg0, sg1, sg2, sg3, ss0, ss1, ss2, ss3, siA, siB, sdA, sdB):
    c = lax.axis_index("c")
    s = lax.axis_index("s")
    gb = [g0, g1, g2, g3]
    sg = [sg0, sg1, sg2, sg3]
    ss = [ss0, ss1, ss2, ss3]
    zeros16 = jnp.zeros((16,), jnp.float32)
    ones16 = jnp.ones((16,), jnp.float32)

    @pl.when((s < 10) & (c == 0))
    def _():
        pltpu.async_copy(x_hbm.at[pl.ds(s * 1000, 1000), pl.ds(0, Dh)],
                         xs_s.at[pl.ds(s * 1000, 1000)], sdA)

    @pl.when((s < 10) & (c == 1))
    def _():
        pltpu.async_copy(x_hbm.at[pl.ds(s * 1000, 1000), pl.ds(Dh, Dh)],
                         xs_s.at[pl.ds(s * 1000, 1000)], sdA)

    pltpu.async_copy(row_hbm.at[pl.ds(s * NCHT, IBLK)], rbA, sdB)
    pltpu.async_copy(col_hbm.at[pl.ds(s * NCHT, IBLK)], cbA, sdB)

    def zrow(r, _):
        def zcol(k, _):
            g0[r, pl.ds(k * 16, 16)] = zeros16
            return 0
        return lax.fori_loop(0, Dh // 16, zcol, 0)
    lax.fori_loop(0, CHUNK, zrow, 0)

    def z1(i, _):
        zbuf[pl.ds(i * 16, 16)] = zeros16
        ones_v[pl.ds(i * 16, 16)] = ones16
        return 0
    lax.fori_loop(0, CHUNK // 16, z1, 0)

    def zagg(k, _):
        pltpu.sync_copy(g0, agg_s.at[pl.ds(s * STRIPE + k * CHUNK, CHUNK)])
        pltpu.sync_copy(zbuf, deg_s.at[pl.ds(s * STRIPE + k * CHUNK, CHUNK)])
        return 0
    lax.fori_loop(0, STRIPE // CHUNK, zagg, 0)

    @pl.when((s < 10) & (c == 0))
    def _():
        pltpu.make_async_copy(x_hbm.at[pl.ds(s * 1000, 1000), pl.ds(0, Dh)],
                              xs_s.at[pl.ds(s * 1000, 1000)], sdA).wait()

    @pl.when((s < 10) & (c == 1))
    def _():
        pltpu.make_async_copy(x_hbm.at[pl.ds(s * 1000, 1000), pl.ds(Dh, Dh)],
                              xs_s.at[pl.ds(s * 1000, 1000)], sdA).wait()

    pltpu.make_async_copy(row_hbm.at[pl.ds(s * NCHT, IBLK)], rbA, sdB).wait()
    pltpu.make_async_copy(col_hbm.at[pl.ds(s * NCHT, IBLK)], cbA, sdB).wait()

    plsc.subcore_barrier()

    base = s * NCHT

    def chunk(l, ph, rb, cb, cbn, first_ever):
        k = (l + ph) % 4
        kp = (k + 3) % 4
        pltpu.make_async_copy(xs_s.at[cb.at[l]], gb[k], sg[k]).wait()
        pltpu.async_copy(gb[k], agg_s.at[rb.at[l]], ss[k], add=True)

        dsem = sdA if k < 2 else sdB
        @pl.when(c == (k % 2))
        def _():
            if not (first_ever and l < 4):
                pltpu.make_async_copy(ones_v, deg_s.at[rb.at[l]], dsem).wait()
            pltpu.async_copy(ones_v, deg_s.at[rb.at[l]], dsem, add=True)

        if not (first_ever and l == 0):
            pltpu.make_async_copy(gb[kp], agg_s.at[rb.at[l]], ss[kp]).wait()
        if l + 3 < IBLK:
            pltpu.async_copy(xs_s.at[cb.at[l + 3]], gb[kp], sg[kp])
        else:
            pltpu.async_copy(xs_s.at[cbn.at[l + 3 - IBLK]], gb[kp], sg[kp])

    def block(t, ph, rb, cb, rbn, cbn, sin, first_ever):
        for l in range(4):
            chunk(l, ph, rb, cb, cbn, first_ever)
        off = base + lax.rem(t + 1, NBLK) * IBLK
        pltpu.async_copy(row_hbm.at[pl.ds(off, IBLK)], rbn, sin)
        pltpu.async_copy(col_hbm.at[pl.ds(off, IBLK)], cbn, sin)
        for l in range(4, IBLK - 3):
            chunk(l, ph, rb, cb, cbn, first_ever)
        pltpu.make_async_copy(row_hbm.at[pl.ds(base, IBLK)], rbn, sin).wait()
        pltpu.make_async_copy(col_hbm.at[pl.ds(base, IBLK)], cbn, sin).wait()
        for l in range(IBLK - 3, IBLK):
            chunk(l, ph, rb, cb, cbn, first_ever)

    for k in range(3):
        pltpu.async_copy(xs_s.at[cbA.at[k]], gb[k], sg[k])

    block(jnp.int32(0), 0, rbA, cbA, rbB, cbB, siB, True)
    block(jnp.int32(1), 2, rbB, cbB, rbA, cbA, siA, False)

    def pair(p, _):
        t = 2 * p
        block(t, 0, rbA, cbA, rbB, cbB, siB, False)
        block(t + 1, 2, rbB, cbB, rbA, cbA, siA, False)
        return 0
    lax.fori_loop(1, NPAIR, pair, 0)

    pltpu.make_async_copy(gb[3], agg_s.at[rbA.at[0]], ss[3]).wait()
    for k in range(3):
        pltpu.make_async_copy(xs_s.at[cbA.at[0]], gb[k], sg[k]).wait()
    pltpu.make_async_copy(ones_v, deg_s.at[rbA.at[0]], sdA).wait()
    pltpu.make_async_copy(ones_v, deg_s.at[rbA.at[0]], sdB).wait()

    @pl.when(s < 4)
    def _():
        pltpu.sync_copy(row_hbm.at[pl.ds(NS * NCHT + s, 1)], rbA.at[pl.ds(0, 1)])
        pltpu.sync_copy(col_hbm.at[pl.ds(NS * NCHT + s, 1)], cbA.at[pl.ds(0, 1)])
        pltpu.sync_copy(xs_s.at[cbA.at[0]], g0)
        pltpu.sync_copy(g0, agg_s.at[rbA.at[0]], add=True)

        @pl.when(c == (s % 2))
        def _():
            pltpu.sync_copy(ones_v, deg_s.at[rbA.at[0]], add=True)

    plsc.subcore_barrier()

    @pl.when(c == 0)
    def _():
        pltpu.sync_copy(agg_s.at[pl.ds(s * STRIPE, STRIPE)],
                        agg0_hbm.at[pl.ds(s * STRIPE, STRIPE)])

    @pl.when(c == 1)
    def _():
        pltpu.sync_copy(agg_s.at[pl.ds(s * STRIPE, STRIPE)],
                        agg1_hbm.at[pl.ds(s * STRIPE, STRIPE)])

    @pl.when((c == 0) & (s == 0))
    def _():
        pltpu.sync_copy(deg_s, deg0_hbm)

    @pl.when((c == 1) & (s == 0))
    def _():
        pltpu.sync_copy(deg_s, deg1_hbm)


def _sc_aggregate(x, row_r, col_r):
    mesh = plsc.VectorSubcoreMesh(
        core_axis_name="c", subcore_axis_name="s", num_cores=NC, num_subcores=NS
    )
    f32 = jnp.float32
    sem = pltpu.SemaphoreType.DMA
    return pl.kernel(
        _sc_body,
        out_type=[
            jax.ShapeDtypeStruct((NP, Dh), f32),
            jax.ShapeDtypeStruct((NP, Dh), f32),
            jax.ShapeDtypeStruct((NP,), f32),
            jax.ShapeDtypeStruct((NP,), f32),
        ],
        mesh=mesh,
        compiler_params=pltpu.CompilerParams(use_tc_tiling_on_sc=False),
        scratch_types=[
            pltpu.VMEM_SHARED((NP, Dh), f32),
            pltpu.VMEM_SHARED((NP, Dh), f32),
            pltpu.VMEM_SHARED((NP,), f32),
            pltpu.VMEM((IBLK, CHUNK), jnp.int32),
            pltpu.VMEM((IBLK, CHUNK), jnp.int32),
            pltpu.VMEM((IBLK, CHUNK), jnp.int32),
            pltpu.VMEM((IBLK, CHUNK), jnp.int32),
            pltpu.VMEM((CHUNK, Dh), f32),
            pltpu.VMEM((CHUNK, Dh), f32),
            pltpu.VMEM((CHUNK, Dh), f32),
            pltpu.VMEM((CHUNK, Dh), f32),
            pltpu.VMEM((CHUNK,), f32),
            pltpu.VMEM((CHUNK,), f32),
            sem, sem, sem, sem,
            sem, sem, sem, sem,
            sem, sem,
            sem, sem,
        ],
    )(x, row_r, col_r)


def _tc_body(a0_ref, a1_ref, d0_ref, d1_ref, w_ref, b_ref, o_ref):
    deg = d0_ref[...] + d1_ref[...]
    dinv = jnp.where(deg > 0.0, 1.0 / deg, 0.0)
    a0 = a0_ref[...] * dinv
    a1 = a1_ref[...] * dinv
    w = w_ref[...]
    dn = (((1,), (1,)), ((), ()))
    o_ref[...] = (
        lax.dot_general(a0, w[:, :Dh], dn, preferred_element_type=jnp.float32)
        + lax.dot_general(a1, w[:, Dh:], dn, preferred_element_type=jnp.float32)
        + b_ref[...]
    )


def _tc_epilogue(agg0, agg1, deg0, deg1, W, b2):
    Bn = 1000
    grid = (N // Bn,)
    return pl.pallas_call(
        _tc_body,
        grid=grid,
        in_specs=[
            pl.BlockSpec((Bn, Dh), lambda i: (i, 0)),
            pl.BlockSpec((Bn, Dh), lambda i: (i, 0)),
            pl.BlockSpec((Bn, 1), lambda i: (i, 0)),
            pl.BlockSpec((Bn, 1), lambda i: (i, 0)),
            pl.BlockSpec((D, D), lambda i: (0, 0)),
            pl.BlockSpec((1, D), lambda i: (0, 0)),
        ],
        out_specs=pl.BlockSpec((Bn, D), lambda i: (i, 0)),
        out_shape=jax.ShapeDtypeStruct((N, D), jnp.float32),
    )(agg0, agg1, deg0, deg1, W, b2)


def kernel(x, edge_index, W, b):
    row_r = edge_index[0].astype(jnp.int32).reshape(ECH, CHUNK)
    col_r = edge_index[1].astype(jnp.int32).reshape(ECH, CHUNK)
    agg0, agg1, deg0, deg1 = _sc_aggregate(x, row_r, col_r)

    return _tc_epilogue(agg0, agg1, deg0.reshape(NP, 1), deg1.reshape(NP, 1),
                        W, b.reshape(1, D))
